# Initial kernel scaffold; baseline (speedup 1.0000x reference)
import jax
import jax.numpy as jnp
from jax.experimental import pallas as pl


def kernel(x, neighbour_indices, W0, b0, W1, b1, W2, b2):
    raise NotImplementedError("write your pallas kernel here")


# speedup vs baseline: 2.9256x; 2.9256x over previous
import functools

import jax
import jax.numpy as jnp
from jax import lax
from jax.experimental import pallas as pl
from jax.experimental.pallas import tpu as pltpu
from jax.experimental.pallas import tpu_sc as plsc

N = 10000
K = 32
F = 64
L = 16
NC, NS = 2, 16
NW = NC * NS
RPW = 320
NP = NW * RPW
CQ = 4
CR = CQ * K
NCH = RPW // CQ
FV = F // L


def _dense_body(x_ref, w_ref, b_ref, o_ref):
    y = jnp.dot(x_ref[...], w_ref[...], preferred_element_type=jnp.float32)
    o_ref[...] = jnp.maximum(y + b_ref[...], 0.0)


def _dense_relu(x, w, b):
    bm = 1024
    din = x.shape[1]
    return pl.pallas_call(
        _dense_body,
        grid=(NP // bm,),
        in_specs=[
            pl.BlockSpec((bm, din), lambda i: (i, 0)),
            pl.BlockSpec((din, F), lambda i: (0, 0)),
            pl.BlockSpec((1, F), lambda i: (0, 0)),
        ],
        out_specs=pl.BlockSpec((bm, F), lambda i: (i, 0)),
        out_shape=jax.ShapeDtypeStruct((NP, F), jnp.float32),
    )(x, w, b.reshape(1, F))


_MESH = plsc.VectorSubcoreMesh(core_axis_name="c", subcore_axis_name="s")


@functools.partial(
    pl.kernel,
    out_type=jax.ShapeDtypeStruct((NP, 2 * F), jnp.float32),
    mesh=_MESH,
    scratch_types=[
        pltpu.VMEM((NCH, CR), jnp.int32),
        pltpu.VMEM((RPW, F), jnp.float32),
        pltpu.VMEM((2, CR, F), jnp.float32),
        pltpu.VMEM((RPW, 2 * F), jnp.float32),
        pltpu.SemaphoreType.DMA,
        pltpu.SemaphoreType.DMA,
    ],
    compiler_params=pltpu.CompilerParams(use_tc_tiling_on_sc=False),
)
def _sc_accum(table_hbm, idx_hbm, out_hbm, idx_v, feat_v, gbuf, out_v, sem0, sem1):
    wid = lax.axis_index("s") * NC + lax.axis_index("c")
    base = wid * RPW
    pltpu.sync_copy(idx_hbm.at[wid], idx_v)
    pltpu.sync_copy(table_hbm.at[pl.ds(base, RPW)], feat_v)
    sems = (sem0, sem1)
    pltpu.async_copy(table_hbm.at[idx_v.at[0]], gbuf.at[0], sem0)
    pltpu.async_copy(table_hbm.at[idx_v.at[1]], gbuf.at[1], sem1)

    @pl.loop(0, NCH, step=2)
    def _ring(g):
        for b in range(2):
            ch = g + b
            sem = sems[b]
            pltpu.make_async_copy(
                table_hbm.at[idx_v.at[ch]], gbuf.at[b], sem).wait()
            for qi in range(CQ):
                row0 = qi * K
                first = tuple(gbuf[b, row0, pl.ds(f * L, L)] for f in range(FV))

                def nbody(n, carry, _b=b, _row0=row0):
                    sums, maxs = carry
                    vals = [gbuf[_b, _row0 + n, pl.ds(f * L, L)]
                            for f in range(FV)]
                    sums = tuple(s + v for s, v in zip(sums, vals))
                    maxs = tuple(jnp.maximum(m, v) for m, v in zip(maxs, vals))
                    return sums, maxs

                sums, maxs = lax.fori_loop(1, K, nbody, (first, first))
                qrow = ch * CQ + qi
                for f in range(FV):
                    fv = feat_v[qrow, pl.ds(f * L, L)]
                    out_v[qrow, pl.ds(f * L, L)] = sums[f] * (1.0 / K) - fv
                    out_v[qrow, pl.ds(F + f * L, L)] = maxs[f] - fv

            @pl.when(ch + 2 < NCH)
            def _prefetch(_b=b, _ch=ch, _sem=sem):
                pltpu.async_copy(
                    table_hbm.at[idx_v.at[_ch + 2]], gbuf.at[_b], _sem)

    pltpu.sync_copy(out_v, out_hbm.at[pl.ds(base, RPW)])


def kernel(x, neighbour_indices, W0, b0, W1, b1, W2, b2):
    xp = jnp.pad(x, ((0, NP - N), (0, 0)))
    idxp = jnp.pad(neighbour_indices.astype(jnp.int32), ((0, NP - N), (0, 0)))
    idx3 = idxp.reshape(NW, NCH, CR)
    prev = xp
    outs = []
    for (W, b) in ((W0, b0), (W1, b1), (W2, b2)):
        feat = _dense_relu(prev, W, b)
        o = _sc_accum(feat, idx3)
        outs.append(o[:N])
        prev = o
    return jnp.concatenate(outs + [x], axis=1)
